# baseline (speedup 1.0000x reference)
import functools

import jax
import jax.numpy as jnp
from jax import lax
from jax.experimental import pallas as pl
from jax.experimental.pallas import tpu as pltpu
from jax.experimental.pallas import tpu_sc as plsc

NC = 2
NS = 16
NW = NC * NS

CH = 256
BBLK = 512
NG = 3
NT = 2
VL = 16


@functools.lru_cache(maxsize=None)
def _make_gather(HIST, BATCH, V, D):
    mesh = plsc.VectorSubcoreMesh(core_axis_name="c", subcore_axis_name="s")
    cpr = BBLK // CH
    n_chunks = HIST * cpr
    assert BATCH == NW * BBLK and D % VL == 0 and CH % VL == 0
    n_pro = NT
    n_epi = NG + 5
    n_main = n_chunks - n_pro - n_epi
    L = 6
    assert n_main % L == 0 and n_main > 0

    @functools.partial(
        pl.kernel,
        out_type=jax.ShapeDtypeStruct(
            (HIST, D // 8, BATCH // 128, 8, 128), jnp.float32),
        mesh=mesh,
        compiler_params=pltpu.CompilerParams(
            use_tc_tiling_on_sc=False, needs_layout_passes=False),
        scratch_types=[
            pltpu.VMEM((HIST, BBLK), jnp.int32),
            [pltpu.VMEM((CH, D), jnp.float32) for _ in range(NG)],
            [pltpu.VMEM((D // 8, CH // 128, 8, 128), jnp.float32)
             for _ in range(NT)],
            pltpu.SemaphoreType.DMA,
            pltpu.SemaphoreType.DMA,
        ],
    )
    def k(idx_hbm, table_hbm, out_hbm, idx_v, rbufs, tbufs, gsem, osem):
        wid = lax.axis_index("s") * NC + lax.axis_index("c")
        b0 = wid * BBLK

        pltpu.sync_copy(idx_hbm.at[:, pl.ds(b0, BBLK)], idx_v)

        jvs = [lax.iota(jnp.int32, VL) + (VL * kk) for kk in range(CH // VL)]

        def hc(g):
            return g // cpr, lax.rem(g, cpr) if not isinstance(g, int) else g % cpr

        def start_gather(g, b):
            h, c = hc(g)
            pltpu.async_copy(
                table_hbm.at[idx_v.at[h, pl.ds(c * CH, CH)]], rbufs[b], gsem)

        def wait_gather(b):
            pltpu.make_async_copy(
                table_hbm.at[idx_v.at[0, pl.ds(0, CH)]], rbufs[b], gsem).wait()

        def transpose(b, t):
            rb, tb = rbufs[b], tbufs[t]

            def body(d, carry):
                dv = jnp.full((VL,), d, dtype=jnp.int32)
                dg = d // 8
                s = lax.rem(d, 8)
                for kk in range(CH // VL):
                    tb[dg, kk // 8, s, pl.ds(VL * (kk % 8), VL)] = (
                        plsc.load_gather(rb, [jvs[kk], dv]))
                return carry

            lax.fori_loop(0, D, body, None, unroll=False)

        def start_out(g, t):
            h, c = hc(g)
            pltpu.async_copy(
                tbufs[t],
                out_hbm.at[h, :, pl.ds((b0 + c * CH) // 128, CH // 128)],
                osem)

        def wait_out():
            pltpu.make_async_copy(
                tbufs[0], out_hbm.at[0, :, pl.ds(0, CH // 128)], osem).wait()

        def step(g, b, t, do_owait, do_gissue):
            wait_gather(b)
            if do_owait:
                wait_out()
            transpose(b, t)
            start_out(g, t)
            if do_gissue:
                start_gather(g + NG, b)

        for g in range(NG):
            start_gather(g, g)

        for g in range(n_pro):
            step(g, g % NG, g % NT, False, True)

        def block(blk, carry):
            gbase = n_pro + blk * L
            for i in range(L):
                gi = n_pro + i
                step(gbase + i, gi % NG, gi % NT, True, True)
            return carry

        lax.fori_loop(0, n_main // L, block, None, unroll=False)

        for g in range(n_chunks - n_epi, n_chunks):
            step(g, g % NG, g % NT, True, g + NG < n_chunks)

        for _ in range(NT):
            wait_out()

    return k


def kernel(x, weight):
    BATCH, HIST = x.shape
    V, D = weight.shape
    idx = x.T.astype(jnp.int32)
    out = _make_gather(HIST, BATCH, V, D)(idx, weight)
    return jnp.transpose(out, (2, 4, 0, 1, 3)).reshape(BATCH, HIST, D)


# speedup vs baseline: 1.7793x; 1.7793x over previous
import functools

import jax
import jax.numpy as jnp
from jax import lax
from jax.experimental import pallas as pl
from jax.experimental.pallas import tpu as pltpu
from jax.experimental.pallas import tpu_sc as plsc

NC = 2
NS = 16
NW = NC * NS

CH = 256
BBLK = 512
NG = 3
NT = 2
VL = 16


@functools.lru_cache(maxsize=None)
def _make_gather(HIST, BATCH, V, D):
    mesh = plsc.VectorSubcoreMesh(core_axis_name="c", subcore_axis_name="s")
    cpr = BBLK // CH
    n_chunks = HIST * cpr
    assert BATCH == NW * BBLK and D % VL == 0 and CH % VL == 0
    n_pro = NT
    n_epi = NG + 5
    n_main = n_chunks - n_pro - n_epi
    L = 6
    assert n_main % L == 0 and n_main > 0

    @functools.partial(
        pl.kernel,
        out_type=jax.ShapeDtypeStruct(
            (HIST, D // 8, BATCH // 128, 8, 128), jnp.float32),
        mesh=mesh,
        compiler_params=pltpu.CompilerParams(
            use_tc_tiling_on_sc=False, needs_layout_passes=False),
        scratch_types=[
            pltpu.VMEM((HIST, BBLK), jnp.int32),
            [pltpu.VMEM((CH, D), jnp.float32) for _ in range(NG)],
            [pltpu.VMEM((D // 8, CH // 128, 8, 128), jnp.float32)
             for _ in range(NT)],
            pltpu.SemaphoreType.DMA,
            pltpu.SemaphoreType.DMA,
        ],
    )
    def k(idx_hbm, table_hbm, out_hbm, idx_v, rbufs, tbufs, gsem, osem):
        wid = lax.axis_index("s") * NC + lax.axis_index("c")
        b0 = wid * BBLK

        pltpu.sync_copy(idx_hbm.at[:, pl.ds(b0, BBLK)], idx_v)

        viota = lax.iota(jnp.int32, VL)

        def hc(g):
            return g // cpr, lax.rem(g, cpr) if not isinstance(g, int) else g % cpr

        def start_gather(g, b):
            h, c = hc(g)
            pltpu.async_copy(
                table_hbm.at[idx_v.at[h, pl.ds(c * CH, CH)]], rbufs[b], gsem)

        def wait_gather(b):
            pltpu.make_async_copy(
                table_hbm.at[idx_v.at[0, pl.ds(0, CH)]], rbufs[b], gsem).wait()

        def transpose(b, t):
            rb, tb = rbufs[b], tbufs[t]

            def body(it, carry):
                j0 = (it // (D // VL)) * VL
                d0 = lax.rem(it, D // VL) * VL
                jv = j0 + viota
                jt = j0 // 128
                jtv = jnp.full((VL,), jt, dtype=jnp.int32)
                lv = lax.rem(j0, 128) + viota
                for r in range(VL):
                    dv = d0 + ((viota + r) & (VL - 1))
                    v = plsc.load_gather(rb, [jv, dv])
                    plsc.store_scatter(
                        tb, [dv // 8, jtv, lax.rem(dv, 8), lv], v)
                return carry

            lax.fori_loop(0, (CH // VL) * (D // VL), body, None,
                          unroll=False)

        def start_out(g, t):
            h, c = hc(g)
            pltpu.async_copy(
                tbufs[t],
                out_hbm.at[h, :, pl.ds((b0 + c * CH) // 128, CH // 128)],
                osem)

        def wait_out():
            pltpu.make_async_copy(
                tbufs[0], out_hbm.at[0, :, pl.ds(0, CH // 128)], osem).wait()

        def step(g, b, t, do_owait, do_gissue):
            wait_gather(b)
            if do_owait:
                wait_out()
            transpose(b, t)
            start_out(g, t)
            if do_gissue:
                start_gather(g + NG, b)

        for g in range(NG):
            start_gather(g, g)

        for g in range(n_pro):
            step(g, g % NG, g % NT, False, True)

        def block(blk, carry):
            gbase = n_pro + blk * L
            for i in range(L):
                gi = n_pro + i
                step(gbase + i, gi % NG, gi % NT, True, True)
            return carry

        lax.fori_loop(0, n_main // L, block, None, unroll=False)

        for g in range(n_chunks - n_epi, n_chunks):
            step(g, g % NG, g % NT, True, g + NG < n_chunks)

        for _ in range(NT):
            wait_out()

    return k


def kernel(x, weight):
    BATCH, HIST = x.shape
    V, D = weight.shape
    idx = x.T.astype(jnp.int32)
    out = _make_gather(HIST, BATCH, V, D)(idx, weight)
    return jnp.transpose(out, (2, 4, 0, 1, 3)).reshape(BATCH, HIST, D)
